# baseline (speedup 1.0000x reference)
import jax
import jax.numpy as jnp
from jax import lax
from jax.experimental import pallas as pl
from jax.experimental.pallas import tpu as pltpu
from jax.experimental.pallas import tpu_sc as plsc

_GRID = 64
_S = 32
_P = 8192
_VDIM = 3
_G3 = _GRID * _GRID * _GRID
_L = 16
_BLK = 128
_GPB = _BLK // _L
_NB = _P // _BLK
_NSTREAM = 8 * _VDIM

_CORNER_OFFS = tuple(
    ((k >> 2) * _GRID * _GRID + ((k >> 1) & 1) * _GRID + (k & 1)) * _VDIM
    for k in range(8))


def _tc_body(p_ref, x_ref, t_ref):
    p = p_ref[0]
    c = p[:, 0:1]
    off = p[:, 1:2]
    sp = p[:, 2:3]
    rad = p[0:1, 3:4]
    d = x_ref[0] - c
    q = d[0:1] * d[0:1] + d[1:2] * d[1:2] + d[2:3] * d[2:3]
    nd = jnp.maximum(jnp.sqrt(q) / rad, 1.0)
    t_ref[0] = (d / nd + off) / sp


def _sc_body(t_hbm, sb_hbm, tab_hbm, out_hbm,
             t_v, sb_v, idx_v, w_v, cv_v, out_v, sem):
    wid = lax.axis_index("s") * 2 + lax.axis_index("c")
    pltpu.sync_copy(t_hbm.at[pl.ds(wid * (3 * _P), 3 * _P)], t_v)
    pltpu.sync_copy(sb_hbm.at[pl.ds(wid * _L, _L)], sb_v)
    base3 = sb_v[pl.ds(0, _L)]

    def block(b, carry):
        bp = b * _BLK
        for g in range(_GPB):
            gp = bp + g * _L
            tx = t_v[pl.ds(gp, _L)]
            ty = t_v[pl.ds(_P + gp, _L)]
            tz = t_v[pl.ds(2 * _P + gp, _L)]
            ix = tx.astype(jnp.int32)
            iy = ty.astype(jnp.int32)
            iz = tz.astype(jnp.int32)
            w_v[pl.ds(g * _L, _L)] = tx - ix.astype(jnp.float32)
            w_v[pl.ds(_BLK + g * _L, _L)] = ty - iy.astype(jnp.float32)
            w_v[pl.ds(2 * _BLK + g * _L, _L)] = tz - iz.astype(jnp.float32)
            rowb3 = base3 + ((ix * _GRID + iy) * _GRID + iz) * _VDIM
            for k in range(8):
                for c in range(_VDIM):
                    idx_v[k * _VDIM + c, pl.ds(g * _L, _L)] = (
                        rowb3 + (_CORNER_OFFS[k] + c))
        descs = []
        for j in range(_NSTREAM):
            descs.append(pltpu.async_copy(
                tab_hbm.at[idx_v.at[j]], cv_v.at[j], sem))
        for d in descs:
            d.wait()
        for g in range(_GPB):
            wx = w_v[pl.ds(g * _L, _L)]
            wy = w_v[pl.ds(_BLK + g * _L, _L)]
            wz = w_v[pl.ds(2 * _BLK + g * _L, _L)]
            ex = 1.0 - wx
            ey = 1.0 - wy
            ez = 1.0 - wz
            wyz = ((ey * ez, ey * wz), (wy * ez, wy * wz))
            acc = [None, None, None]
            for k in range(8):
                wk = (wx if (k >> 2) else ex) * wyz[(k >> 1) & 1][k & 1]
                for c in range(_VDIM):
                    v = cv_v[k * _VDIM + c, pl.ds(g * _L, _L)]
                    acc[c] = wk * v if acc[c] is None else acc[c] + wk * v
            for c in range(_VDIM):
                out_v[pl.ds(c * _P + bp + g * _L, _L)] = acc[c]
        return carry

    lax.fori_loop(0, _NB, block, 0)
    pltpu.sync_copy(out_v, out_hbm.at[pl.ds(wid * (3 * _P), 3 * _P)])


def kernel(x, s, values, center, radius, bounding_box, spacing):
    si = s.astype(jnp.int32)
    c = center[si]
    r = radius[si]
    bb0 = bounding_box[si, 0]
    sp = spacing[si]
    params = jnp.concatenate(
        [jnp.stack([c, c - bb0, sp, jnp.broadcast_to(r[:, None], (_S, 3))],
                   axis=2),
         jnp.zeros((_S, 3, 12), jnp.float32)], axis=2)
    xt = x.transpose(0, 2, 1)
    t = pl.pallas_call(
        _tc_body,
        grid=(_S,),
        in_specs=[pl.BlockSpec((1, 3, 16), lambda i: (i, 0, 0)),
                  pl.BlockSpec((1, 3, _P), lambda i: (i, 0, 0))],
        out_specs=pl.BlockSpec((1, 3, _P), lambda i: (i, 0, 0)),
        out_shape=jax.ShapeDtypeStruct((_S, 3, _P), jnp.float32),
    )(params, xt)
    sb = jnp.broadcast_to((si * (_G3 * _VDIM))[:, None], (_S, _L))
    fn = pl.kernel(
        _sc_body,
        out_type=jax.ShapeDtypeStruct((_S * 3 * _P,), jnp.float32),
        mesh=plsc.VectorSubcoreMesh(core_axis_name="c", subcore_axis_name="s"),
        scratch_types=[
            pltpu.VMEM((3 * _P,), jnp.float32),
            pltpu.VMEM((_L,), jnp.int32),
            pltpu.VMEM((_NSTREAM, _BLK), jnp.int32),
            pltpu.VMEM((3 * _BLK,), jnp.float32),
            pltpu.VMEM((_NSTREAM, _BLK), jnp.float32),
            pltpu.VMEM((3 * _P,), jnp.float32),
            pltpu.SemaphoreType.DMA,
        ],
    )
    out_t = fn(t.reshape(_S * 3 * _P), sb.reshape(_S * _L),
               values.reshape(_S * _G3 * _VDIM))
    return out_t.reshape(_S, 3, _P).transpose(0, 2, 1)


# speedup vs baseline: 1.0137x; 1.0137x over previous
import jax
import jax.numpy as jnp
from jax import lax
from jax.experimental import pallas as pl
from jax.experimental.pallas import tpu as pltpu
from jax.experimental.pallas import tpu_sc as plsc

_GRID = 64
_S = 32
_P = 8192
_VDIM = 3
_G3 = _GRID * _GRID * _GRID
_GW = _G3 * _VDIM
_L = 16
_NT = 16
_RPC = _S // 2
_PT = _P // _NT
_HLF = _PT // 2
_GPH = _HLF // _L
_SL = _GW // _NT
_NSTREAM = 8 * _VDIM

_CORNER_OFFS = tuple(
    ((k >> 2) * _GRID * _GRID + ((k >> 1) & 1) * _GRID + (k & 1)) * _VDIM
    for k in range(8))


def _tc_body(p_ref, x_ref, t_ref):
    p = p_ref[0]
    c = p[:, 0:1]
    off = p[:, 1:2]
    sp = p[:, 2:3]
    rad = p[0:1, 3:4]
    d = x_ref[0] - c
    q = d[0:1] * d[0:1] + d[1:2] * d[1:2] + d[2:3] * d[2:3]
    nd = jnp.maximum(jnp.sqrt(q) / rad, 1.0)
    t_ref[0] = (d / nd + off) / sp


def _sc_body(t_hbm, tab_hbm, out_hbm,
             t_v, idx_v, w_v, cv_v, out_v, spm_a, spm_b, gsem, ssem):
    core = lax.axis_index("c")
    sid = lax.axis_index("s")
    shbase = core * _RPC

    def stage_issue(r_next, spm):
        src = tab_hbm.at[pl.ds((shbase + r_next) * _GW + sid * _SL, _SL)]
        pltpu.async_copy(src, spm.at[pl.ds(sid * _SL, _SL)], ssem)

    def stage_drain(r_next, spm):
        src = tab_hbm.at[pl.ds((shbase + r_next) * _GW + sid * _SL, _SL)]
        pltpu.make_async_copy(
            src, spm.at[pl.ds(sid * _SL, _SL)], ssem).wait()

    def round_body(r, spm, spm_next):
        sh = shbase + r
        tbase = sh * (3 * _P) + sid * _PT

        @pl.when(r + 1 < _RPC)
        def _():
            stage_issue(r + 1, spm_next)

        for comp in range(3):
            pltpu.sync_copy(t_hbm.at[pl.ds(tbase + comp * _P, _PT)],
                            t_v.at[pl.ds(comp * _PT, _PT)])

        def half(h, carry):
            hb = h * _HLF
            for g in range(_GPH):
                gp = hb + g * _L
                tx = t_v[pl.ds(gp, _L)]
                ty = t_v[pl.ds(_PT + gp, _L)]
                tz = t_v[pl.ds(2 * _PT + gp, _L)]
                ix = tx.astype(jnp.int32)
                iy = ty.astype(jnp.int32)
                iz = tz.astype(jnp.int32)
                w_v[pl.ds(g * _L, _L)] = tx - ix.astype(jnp.float32)
                w_v[pl.ds(_HLF + g * _L, _L)] = ty - iy.astype(jnp.float32)
                w_v[pl.ds(2 * _HLF + g * _L, _L)] = tz - iz.astype(jnp.float32)
                rowe = ((ix * _GRID + iy) * _GRID + iz) * _VDIM
                for k in range(8):
                    for c in range(_VDIM):
                        idx_v[pl.ds((k * _VDIM + c) * _HLF + g * _L, _L)] = (
                            rowe + (_CORNER_OFFS[k] + c))
            descs = []
            for j in range(_NSTREAM):
                descs.append(pltpu.async_copy(
                    spm.at[idx_v.at[pl.ds(j * _HLF, _HLF)]],
                    cv_v.at[pl.ds(j * _HLF, _HLF)], gsem))
            for d in descs:
                d.wait()
            for g in range(_GPH):
                wx = w_v[pl.ds(g * _L, _L)]
                wy = w_v[pl.ds(_HLF + g * _L, _L)]
                wz = w_v[pl.ds(2 * _HLF + g * _L, _L)]
                ex = 1.0 - wx
                ey = 1.0 - wy
                ez = 1.0 - wz
                wyz = ((ey * ez, ey * wz), (wy * ez, wy * wz))
                acc = [None, None, None]
                for k in range(8):
                    wk = (wx if (k >> 2) else ex) * wyz[(k >> 1) & 1][k & 1]
                    for c in range(_VDIM):
                        v = cv_v[pl.ds((k * _VDIM + c) * _HLF + g * _L, _L)]
                        acc[c] = wk * v if acc[c] is None else acc[c] + wk * v
                for c in range(_VDIM):
                    out_v[pl.ds(c * _PT + hb + g * _L, _L)] = acc[c]
            return carry

        lax.fori_loop(0, 2, half, 0)

        for comp in range(3):
            pltpu.sync_copy(out_v.at[pl.ds(comp * _PT, _PT)],
                            out_hbm.at[pl.ds(tbase + comp * _P, _PT)])

        @pl.when(r + 1 < _RPC)
        def _():
            stage_drain(r + 1, spm_next)

        plsc.subcore_barrier()

    stage_issue(0, spm_a)
    stage_drain(0, spm_a)
    plsc.subcore_barrier()

    def superround(sr, carry):
        round_body(2 * sr, spm_a, spm_b)
        round_body(2 * sr + 1, spm_b, spm_a)
        return carry

    lax.fori_loop(0, _RPC // 2, superround, 0)


def kernel(x, s, values, center, radius, bounding_box, spacing):
    si = s.astype(jnp.int32)
    c = center[si]
    r = radius[si]
    bb0 = bounding_box[si, 0]
    sp = spacing[si]
    params = jnp.concatenate(
        [jnp.stack([c, c - bb0, sp, jnp.broadcast_to(r[:, None], (_S, 3))],
                   axis=2),
         jnp.zeros((_S, 3, 12), jnp.float32)], axis=2)
    xt = x.transpose(0, 2, 1)
    t = pl.pallas_call(
        _tc_body,
        grid=(_S,),
        in_specs=[pl.BlockSpec((1, 3, 16), lambda i: (i, 0, 0)),
                  pl.BlockSpec((1, 3, _P), lambda i: (i, 0, 0))],
        out_specs=pl.BlockSpec((1, 3, _P), lambda i: (i, 0, 0)),
        out_shape=jax.ShapeDtypeStruct((_S, 3, _P), jnp.float32),
    )(params, xt)
    fn = pl.kernel(
        _sc_body,
        out_type=jax.ShapeDtypeStruct((_S * 3 * _P,), jnp.float32),
        mesh=plsc.VectorSubcoreMesh(core_axis_name="c", subcore_axis_name="s"),
        scratch_types=[
            pltpu.VMEM((3 * _PT,), jnp.float32),
            pltpu.VMEM((_NSTREAM * _HLF,), jnp.int32),
            pltpu.VMEM((3 * _HLF,), jnp.float32),
            pltpu.VMEM((_NSTREAM * _HLF,), jnp.float32),
            pltpu.VMEM((3 * _PT,), jnp.float32),
            pltpu.VMEM_SHARED((_GW,), jnp.float32),
            pltpu.VMEM_SHARED((_GW,), jnp.float32),
            pltpu.SemaphoreType.DMA,
            pltpu.SemaphoreType.DMA,
        ],
    )
    out_t = fn(t.reshape(_S * 3 * _P), values.reshape(_S * _GW))
    return out_t.reshape(_S, 3, _P).transpose(0, 2, 1)


# speedup vs baseline: 1.0143x; 1.0005x over previous
import jax
import jax.numpy as jnp
from jax import lax
from jax.experimental import pallas as pl
from jax.experimental.pallas import tpu as pltpu
from jax.experimental.pallas import tpu_sc as plsc

_GRID = 64
_S = 32
_P = 8192
_VDIM = 3
_G3 = _GRID * _GRID * _GRID
_GW = _G3 * _VDIM
_L = 16
_NT = 16
_RPC = _S // 2
_PT = _P // _NT
_HLF = _PT // 2
_GPH = _HLF // _L
_SL = _GW // _NT
_NSTREAM = 8 * _VDIM

_CORNER_OFFS = tuple(
    ((k >> 2) * _GRID * _GRID + ((k >> 1) & 1) * _GRID + (k & 1)) * _VDIM
    for k in range(8))


def _tc_body(p_ref, x_ref, t_ref):
    p = p_ref[0]
    c = p[:, 0:1]
    off = p[:, 1:2]
    sp = p[:, 2:3]
    rad = p[0:1, 3:4]
    d = x_ref[0] - c
    q = d[0:1] * d[0:1] + d[1:2] * d[1:2] + d[2:3] * d[2:3]
    nd = jnp.maximum(jnp.sqrt(q) / rad, 1.0)
    t_ref[0] = (d / nd + off) / sp


def _sc_body(t_hbm, tab_hbm, out_hbm,
             t_v, idx_v, w_v, cv_v, out_v, spm_a, spm_b, gsem, ssem):
    core = lax.axis_index("c")
    sid = lax.axis_index("s")
    shbase = core * _RPC

    def stage_issue(r_next, spm):
        src = tab_hbm.at[pl.ds((shbase + r_next) * _GW + sid * _SL, _SL)]
        pltpu.async_copy(src, spm.at[pl.ds(sid * _SL, _SL)], ssem)

    def stage_drain(r_next, spm):
        src = tab_hbm.at[pl.ds((shbase + r_next) * _GW + sid * _SL, _SL)]
        pltpu.make_async_copy(
            src, spm.at[pl.ds(sid * _SL, _SL)], ssem).wait()

    def round_body(r, spm, spm_next):
        sh = shbase + r
        tbase = sh * (3 * _P) + sid * _PT

        @pl.when(r + 1 < _RPC)
        def _():
            stage_issue(r + 1, spm_next)

        for comp in range(3):
            pltpu.sync_copy(t_hbm.at[pl.ds(tbase + comp * _P, _PT)],
                            t_v.at[pl.ds(comp * _PT, _PT)])

        def half(h, carry):
            hb = h * _HLF
            for g in range(_GPH):
                gp = hb + g * _L
                tx = t_v[pl.ds(gp, _L)]
                ty = t_v[pl.ds(_PT + gp, _L)]
                tz = t_v[pl.ds(2 * _PT + gp, _L)]
                ix = tx.astype(jnp.int32)
                iy = ty.astype(jnp.int32)
                iz = tz.astype(jnp.int32)
                w_v[pl.ds(g * _L, _L)] = tx - ix.astype(jnp.float32)
                w_v[pl.ds(_HLF + g * _L, _L)] = ty - iy.astype(jnp.float32)
                w_v[pl.ds(2 * _HLF + g * _L, _L)] = tz - iz.astype(jnp.float32)
                rowe = ((ix * _GRID + iy) * _GRID + iz) * _VDIM
                for k in range(8):
                    for c in range(_VDIM):
                        idx_v[pl.ds((k * _VDIM + c) * _HLF + g * _L, _L)] = (
                            rowe + (_CORNER_OFFS[k] + c))
            pltpu.async_copy(spm.at[idx_v.at[...]], cv_v, gsem).wait()
            for g in range(_GPH):
                wx = w_v[pl.ds(g * _L, _L)]
                wy = w_v[pl.ds(_HLF + g * _L, _L)]
                wz = w_v[pl.ds(2 * _HLF + g * _L, _L)]
                ex = 1.0 - wx
                ey = 1.0 - wy
                ez = 1.0 - wz
                wyz = ((ey * ez, ey * wz), (wy * ez, wy * wz))
                acc = [None, None, None]
                for k in range(8):
                    wk = (wx if (k >> 2) else ex) * wyz[(k >> 1) & 1][k & 1]
                    for c in range(_VDIM):
                        v = cv_v[pl.ds((k * _VDIM + c) * _HLF + g * _L, _L)]
                        acc[c] = wk * v if acc[c] is None else acc[c] + wk * v
                for c in range(_VDIM):
                    out_v[pl.ds(c * _PT + hb + g * _L, _L)] = acc[c]
            return carry

        lax.fori_loop(0, 2, half, 0)

        for comp in range(3):
            pltpu.sync_copy(out_v.at[pl.ds(comp * _PT, _PT)],
                            out_hbm.at[pl.ds(tbase + comp * _P, _PT)])

        @pl.when(r + 1 < _RPC)
        def _():
            stage_drain(r + 1, spm_next)

        plsc.subcore_barrier()

    stage_issue(0, spm_a)
    stage_drain(0, spm_a)
    plsc.subcore_barrier()

    def superround(sr, carry):
        round_body(2 * sr, spm_a, spm_b)
        round_body(2 * sr + 1, spm_b, spm_a)
        return carry

    lax.fori_loop(0, _RPC // 2, superround, 0)


def kernel(x, s, values, center, radius, bounding_box, spacing):
    si = s.astype(jnp.int32)
    c = center[si]
    r = radius[si]
    bb0 = bounding_box[si, 0]
    sp = spacing[si]
    params = jnp.concatenate(
        [jnp.stack([c, c - bb0, sp, jnp.broadcast_to(r[:, None], (_S, 3))],
                   axis=2),
         jnp.zeros((_S, 3, 12), jnp.float32)], axis=2)
    xt = x.transpose(0, 2, 1)
    t = pl.pallas_call(
        _tc_body,
        grid=(_S,),
        in_specs=[pl.BlockSpec((1, 3, 16), lambda i: (i, 0, 0)),
                  pl.BlockSpec((1, 3, _P), lambda i: (i, 0, 0))],
        out_specs=pl.BlockSpec((1, 3, _P), lambda i: (i, 0, 0)),
        out_shape=jax.ShapeDtypeStruct((_S, 3, _P), jnp.float32),
    )(params, xt)
    fn = pl.kernel(
        _sc_body,
        out_type=jax.ShapeDtypeStruct((_S * 3 * _P,), jnp.float32),
        mesh=plsc.VectorSubcoreMesh(core_axis_name="c", subcore_axis_name="s"),
        scratch_types=[
            pltpu.VMEM((3 * _PT,), jnp.float32),
            pltpu.VMEM((_NSTREAM * _HLF,), jnp.int32),
            pltpu.VMEM((3 * _HLF,), jnp.float32),
            pltpu.VMEM((_NSTREAM * _HLF,), jnp.float32),
            pltpu.VMEM((3 * _PT,), jnp.float32),
            pltpu.VMEM_SHARED((_GW,), jnp.float32),
            pltpu.VMEM_SHARED((_GW,), jnp.float32),
            pltpu.SemaphoreType.DMA,
            pltpu.SemaphoreType.DMA,
        ],
    )
    out_t = fn(t.reshape(_S * 3 * _P), values.reshape(_S * _GW))
    return out_t.reshape(_S, 3, _P).transpose(0, 2, 1)
